# baseline (speedup 1.0000x reference)
import jax
import jax.numpy as jnp
from jax import lax
from jax.experimental import pallas as pl
from jax.experimental.pallas import tpu as pltpu
from jax.experimental.pallas import tpu_sc as plsc

_VOCAB = 1_000_000
_D = 32
_BATCH = 16384
_FIELDS = 26
_B_TOTAL = _BATCH * _FIELDS
_QUARTER = _VOCAB // 4
_OBAND = _B_TOTAL // 4
_NC, _NS = 2, 16
_NW = _NC * _NS
_B_PER_W = _B_TOTAL // _NW
_CHUNK = 128
_N_CHUNKS = _B_PER_W // _CHUNK
_NBUF = 4
_N_GROUPS = _N_CHUNKS // _NBUF


_PACK_BN = 2000
_PACK_G = _QUARTER // _PACK_BN


def _pack_body(x0, x1, x2, x3, y_ref):
    for k, x in enumerate((x0, x1, x2, x3)):
        y_ref[:, k * _D:(k + 1) * _D] = x[...]


_tc_pack = pl.pallas_call(
    _pack_body,
    grid=(_PACK_G,),
    in_specs=[
        pl.BlockSpec((_PACK_BN, _D), lambda i, k=k: (i + k * _PACK_G, 0))
        for k in range(4)
    ],
    out_specs=pl.BlockSpec((_PACK_BN, 128), lambda i: (i, 0)),
    out_shape=jax.ShapeDtypeStruct((_QUARTER, 128), jnp.float32),
)


_mesh = plsc.VectorSubcoreMesh(
    core_axis_name="c", subcore_axis_name="s", num_cores=_NC, num_subcores=_NS
)


def _gather_body(w2_hbm, q_hbm, off_hbm, out_hbm, q_v, off_v, pbuf, wb, sem_g, sem_w):
    wid = lax.axis_index("s") * _NC + lax.axis_index("c")
    pltpu.sync_copy(q_hbm.at[wid], q_v)
    pltpu.sync_copy(off_hbm.at[wid], off_v)
    row_base = (wid & 7) * _B_PER_W

    for b in range(_NBUF):
        pltpu.async_copy(w2_hbm.at[q_v.at[b]], pbuf.at[b], sem_g.at[b])

    @pl.loop(0, _N_GROUPS)
    def _(g):
        for b in range(_NBUF):
            c = g * _NBUF + b
            wslot = b & 1
            pltpu.make_async_copy(
                w2_hbm.at[q_v.at[c]], pbuf.at[b], sem_g.at[b]
            ).wait()

            @pl.when(c >= 2)
            def _():
                pltpu.make_async_copy(
                    wb.at[wslot],
                    out_hbm.at[pl.ds(0, _CHUNK), pl.ds(0, _D)],
                    sem_w.at[wslot],
                ).wait()

            @pl.loop(0, _CHUNK // 16)
            def _(rg):
                off16 = off_v[c, pl.ds(rg * 16, 16)]
                for k in range(16):
                    src_off = off16[k]
                    for h in range(2):
                        wb[wslot, rg * 16 + k, pl.ds(16 * h, 16)] = (
                            pbuf[b, rg * 16 + k, pl.ds(src_off + 16 * h, 16)]
                        )

            for band in range(4):
                @pl.when(wid >> 3 == band)
                def _():
                    pltpu.async_copy(
                        wb.at[wslot],
                        out_hbm.at[
                            pl.ds(row_base + c * _CHUNK, _CHUNK),
                            pl.ds(band * _D, _D),
                        ],
                        sem_w.at[wslot],
                    )

            @pl.when(c + _NBUF < _N_CHUNKS)
            def _():
                pltpu.async_copy(
                    w2_hbm.at[q_v.at[c + _NBUF]], pbuf.at[b], sem_g.at[b]
                )

    for wslot in range(2):
        pltpu.make_async_copy(
            wb.at[wslot],
            out_hbm.at[pl.ds(0, _CHUNK), pl.ds(0, _D)],
            sem_w.at[wslot],
        ).wait()


_sc_gather = pl.kernel(
    _gather_body,
    out_type=jax.ShapeDtypeStruct((_OBAND, 128), jnp.float32),
    mesh=_mesh,
    scratch_types=[
        pltpu.VMEM((_N_CHUNKS, _CHUNK), jnp.int32),
        pltpu.VMEM((_N_CHUNKS, _CHUNK), jnp.int32),
        pltpu.VMEM((_NBUF, _CHUNK, 128), jnp.float32),
        pltpu.VMEM((2, _CHUNK, _D), jnp.float32),
        pltpu.SemaphoreType.DMA((_NBUF,)),
        pltpu.SemaphoreType.DMA((2,)),
    ],
    compiler_params=pltpu.CompilerParams(use_tc_tiling_on_sc=False),
)


_UNPACK_BB = 128
_UNPACK_BR = _UNPACK_BB * _FIELDS
_UNPACK_G = _BATCH // _UNPACK_BB


def _unpack_body(x_ref, y_ref):
    band = pl.program_id(0) // 32
    x = x_ref[...]
    sel = x[:, 0:_D]
    for k in range(1, 4):
        sel = jnp.where(band == k, x[:, k * _D:(k + 1) * _D], sel)
    y_ref[...] = sel.reshape(_UNPACK_BB, _FIELDS, _D)


_tc_unpack = pl.pallas_call(
    _unpack_body,
    grid=(_UNPACK_G,),
    in_specs=[
        pl.BlockSpec((_UNPACK_BR, 128), lambda i: (i % 32, 0))
    ],
    out_specs=pl.BlockSpec(
        (_UNPACK_BB, _FIELDS, _D), lambda i: (i, 0, 0)
    ),
    out_shape=jax.ShapeDtypeStruct((_BATCH, _FIELDS, _D), jnp.float32),
)


def kernel(weight, input):
    idx = input.astype(jnp.int32).reshape(_NW, _N_CHUNKS, _CHUNK)
    q = idx % _QUARTER
    off = (idx // _QUARTER) * _D
    w2 = jnp.concatenate(
        [weight[k * _QUARTER:(k + 1) * _QUARTER] for k in range(4)], axis=1
    )
    out2 = _sc_gather(w2, q, off)
    return _tc_unpack(out2)


# speedup vs baseline: 1.1125x; 1.1125x over previous
import jax
import jax.numpy as jnp
from jax import lax
from jax.experimental import pallas as pl
from jax.experimental.pallas import tpu as pltpu
from jax.experimental.pallas import tpu_sc as plsc

_VOCAB = 1_000_000
_D = 32
_BATCH = 16384
_FIELDS = 26
_B_TOTAL = _BATCH * _FIELDS
_QUARTER = _VOCAB // 4
_OBAND = _B_TOTAL // 4
_NC, _NS = 2, 16
_NW = _NC * _NS
_B_PER_W = _B_TOTAL // _NW
_CHUNK = 128
_N_CHUNKS = _B_PER_W // _CHUNK
_NBUF = 4
_N_GROUPS = _N_CHUNKS // _NBUF


_PACK_BN = 2000
_PACK_G = _QUARTER // _PACK_BN


def _pack_body(x0, x1, x2, x3, y_ref):
    for k, x in enumerate((x0, x1, x2, x3)):
        y_ref[:, k * _D:(k + 1) * _D] = x[...]


_tc_pack = pl.pallas_call(
    _pack_body,
    grid=(_PACK_G,),
    in_specs=[
        pl.BlockSpec((_PACK_BN, _D), lambda i, k=k: (i + k * _PACK_G, 0))
        for k in range(4)
    ],
    out_specs=pl.BlockSpec((_PACK_BN, 128), lambda i: (i, 0)),
    out_shape=jax.ShapeDtypeStruct((_QUARTER, 128), jnp.float32),
)


_mesh = plsc.VectorSubcoreMesh(
    core_axis_name="c", subcore_axis_name="s", num_cores=_NC, num_subcores=_NS
)


def _gather_body(w2_hbm, q_hbm, off_hbm, out_hbm, q_v, off_v, pbuf, wb, sem_g, sem_w):
    wid = lax.axis_index("s") * _NC + lax.axis_index("c")
    pltpu.sync_copy(q_hbm.at[wid], q_v)
    pltpu.sync_copy(off_hbm.at[wid], off_v)
    row_base = (wid & 7) * _B_PER_W

    for b in range(_NBUF):
        pltpu.async_copy(w2_hbm.at[q_v.at[b]], pbuf.at[b], sem_g.at[b])

    @pl.loop(0, _N_GROUPS)
    def _(g):
        for b in range(_NBUF):
            c = g * _NBUF + b
            wslot = b & 1
            pltpu.make_async_copy(
                w2_hbm.at[q_v.at[c]], pbuf.at[b], sem_g.at[b]
            ).wait()

            @pl.when(c >= 2)
            def _():
                pltpu.make_async_copy(
                    wb.at[wslot],
                    out_hbm.at[pl.ds(0, _CHUNK), pl.ds(0, _D)],
                    sem_w.at[wslot],
                ).wait()

            @pl.loop(0, _CHUNK // 16)
            def _(rg):
                off16 = off_v[c, pl.ds(rg * 16, 16)]
                for k in range(16):
                    src_off = off16[k]
                    for h in range(2):
                        wb[wslot, rg * 16 + k, pl.ds(16 * h, 16)] = (
                            pbuf[b, rg * 16 + k, pl.ds(src_off + 16 * h, 16)]
                        )

            for band in range(4):
                @pl.when(wid >> 3 == band)
                def _():
                    pltpu.async_copy(
                        wb.at[wslot],
                        out_hbm.at[
                            pl.ds(row_base + c * _CHUNK, _CHUNK),
                            pl.ds(band * _D, _D),
                        ],
                        sem_w.at[wslot],
                    )

            @pl.when(c + _NBUF < _N_CHUNKS)
            def _():
                pltpu.async_copy(
                    w2_hbm.at[q_v.at[c + _NBUF]], pbuf.at[b], sem_g.at[b]
                )

    for wslot in range(2):
        pltpu.make_async_copy(
            wb.at[wslot],
            out_hbm.at[pl.ds(0, _CHUNK), pl.ds(0, _D)],
            sem_w.at[wslot],
        ).wait()


_sc_gather = pl.kernel(
    _gather_body,
    out_type=jax.ShapeDtypeStruct((_OBAND, 128), jnp.float32),
    mesh=_mesh,
    scratch_types=[
        pltpu.VMEM((_N_CHUNKS, _CHUNK), jnp.int32),
        pltpu.VMEM((_N_CHUNKS, _CHUNK), jnp.int32),
        pltpu.VMEM((_NBUF, _CHUNK, 128), jnp.float32),
        pltpu.VMEM((2, _CHUNK, _D), jnp.float32),
        pltpu.SemaphoreType.DMA((_NBUF,)),
        pltpu.SemaphoreType.DMA((2,)),
    ],
    compiler_params=pltpu.CompilerParams(use_tc_tiling_on_sc=False),
)


_UNPACK_BB = 128
_UNPACK_BR = _UNPACK_BB * _FIELDS
_UNPACK_G = _BATCH // _UNPACK_BB


def _unpack_body(x_ref, y_ref):
    band = pl.program_id(0) // 32
    x = x_ref[...]
    sel = x[:, 0:_D]
    for k in range(1, 4):
        sel = jnp.where(band == k, x[:, k * _D:(k + 1) * _D], sel)
    y_ref[...] = sel.reshape(_UNPACK_BB, _FIELDS, _D)


_tc_unpack = pl.pallas_call(
    _unpack_body,
    grid=(_UNPACK_G,),
    in_specs=[
        pl.BlockSpec((_UNPACK_BR, 128), lambda i: (i % 32, 0))
    ],
    out_specs=pl.BlockSpec(
        (_UNPACK_BB, _FIELDS, _D), lambda i: (i, 0, 0)
    ),
    out_shape=jax.ShapeDtypeStruct((_BATCH, _FIELDS, _D), jnp.float32),
)


def kernel(weight, input):
    idx = input.astype(jnp.int32).reshape(_NW, _N_CHUNKS, _CHUNK)
    q = idx >> 2
    off = (idx & 3) * _D
    w2 = weight.reshape(_QUARTER, 128)
    out2 = _sc_gather(w2, q, off)
    return _tc_unpack(out2)


# speedup vs baseline: 1.5248x; 1.3706x over previous
import jax
import jax.numpy as jnp
from jax import lax
from jax.experimental import pallas as pl
from jax.experimental.pallas import tpu as pltpu
from jax.experimental.pallas import tpu_sc as plsc

_VOCAB = 1_000_000
_D = 32
_BATCH = 16384
_FIELDS = 26
_B_TOTAL = _BATCH * _FIELDS
_NC, _NS = 2, 16
_NW = _NC * _NS
_B_PER_W = _B_TOTAL // _NW
_CHUNK = 128
_N_CHUNKS = _B_PER_W // _CHUNK
_NBUF = 4
_N_GROUPS = _N_CHUNKS // _NBUF

_mesh = plsc.VectorSubcoreMesh(
    core_axis_name="c", subcore_axis_name="s", num_cores=_NC, num_subcores=_NS
)


def _gather_body(table_hbm, idx_hbm, out_hbm, idx_v, rows_v, sems):
    wid = lax.axis_index("s") * _NC + lax.axis_index("c")
    pltpu.sync_copy(idx_hbm.at[wid], idx_v)

    for b in range(_NBUF):
        pltpu.async_copy(table_hbm.at[idx_v.at[b]], rows_v.at[b], sems.at[b])

    @pl.loop(0, _N_GROUPS)
    def _(g):
        for b in range(_NBUF):
            j = g * _NBUF + b
            pltpu.make_async_copy(
                table_hbm.at[idx_v.at[j]], rows_v.at[b], sems.at[b]
            ).wait()
            base = wid * _B_PER_W + j * _CHUNK
            pltpu.sync_copy(rows_v.at[b], out_hbm.at[pl.ds(base, _CHUNK)])
            nj = j + _NBUF

            @pl.when(nj < _N_CHUNKS)
            def _():
                pltpu.async_copy(
                    table_hbm.at[idx_v.at[nj]], rows_v.at[b], sems.at[b]
                )


_gather = pl.kernel(
    _gather_body,
    out_type=jax.ShapeDtypeStruct((_B_TOTAL, _D), jnp.float32),
    mesh=_mesh,
    scratch_types=[
        pltpu.VMEM((_N_CHUNKS, _CHUNK), jnp.int32),
        pltpu.VMEM((_NBUF, _CHUNK, _D), jnp.float32),
        pltpu.SemaphoreType.DMA((_NBUF,)),
    ],
    compiler_params=pltpu.CompilerParams(use_tc_tiling_on_sc=False),
)


def kernel(weight, input):
    idx = input.astype(jnp.int32).reshape(_NW, _N_CHUNKS, _CHUNK)
    flat = _gather(weight, idx)
    return flat.reshape(_BATCH, _FIELDS, _D)
